# Initial kernel scaffold; baseline (speedup 1.0000x reference)
import jax
import jax.numpy as jnp
from jax.experimental import pallas as pl


def kernel(edge_dist, node_emb, edge_W, edge_b, msg_W1, msg_b1, msg_W2, msg_b2, soft_W, soft_b, upd_W1, upd_b1, upd_W2, upd_b2, out1_W, out1_b, out2_W, out2_b, ro1_W, ro1_b, ro2_W, ro2_b, edge_index):
    raise NotImplementedError("write your pallas kernel here")


# speedup vs baseline: 3.6450x; 3.6450x over previous
import functools

import jax
import jax.numpy as jnp
from jax import lax
from jax.experimental import pallas as pl
from jax.experimental.pallas import tpu as pltpu
from jax.experimental.pallas import tpu_sc as plsc

N = 10000
E = 320000
H = 128
L = 4
TARGET = 32

NC = 2
NS = 16
NW = NC * NS
EPW = E // NW
CH = 80
NCH = EPW // CH
NPS = 624
NTL = N - NS * NPS

EB = 4000
NB = 2000

_silu = jax.nn.silu
_sigmoid = jax.nn.sigmoid

def _vmesh():
  return plsc.VectorSubcoreMesh(core_axis_name="c", subcore_axis_name="s",
                                num_cores=NC, num_subcores=NS)


def _gather_body(a_hbm, b_hbm, src_hbm, dst_hbm, g_hbm,
                 srcv, dstv, buf_a, buf_b, sem_a, sem_b):
  c = lax.axis_index("c")
  s = lax.axis_index("s")
  w = s * NC + c
  pltpu.sync_copy(src_hbm.at[w], srcv)
  pltpu.sync_copy(dst_hbm.at[w], dstv)

  def chunk(j, _):
    ca = pltpu.async_copy(a_hbm.at[srcv.at[j]], buf_a, sem_a)
    cb = pltpu.async_copy(b_hbm.at[dstv.at[j]], buf_b, sem_b)
    ca.wait()
    cb.wait()

    def add_row(r, _):
      for q in range(H // 16):
        x = buf_b[r, pl.ds(q * 16, 16)]
        plsc.addupdate(buf_a.at[r, pl.ds(q * 16, 16)], x)
      return 0

    lax.fori_loop(0, CH, add_row, 0)
    pltpu.sync_copy(buf_a, g_hbm.at[pl.ds(w * EPW + j * CH, CH)])
    return 0

  lax.fori_loop(0, NCH, chunk, 0)


def _sc_gather(a, b, src3, dst3):
  k = pl.kernel(
      _gather_body,
      out_type=jax.ShapeDtypeStruct((E, H), jnp.float32),
      mesh=_vmesh(),
      scratch_types=[
          pltpu.VMEM((NCH, CH), jnp.int32),
          pltpu.VMEM((NCH, CH), jnp.int32),
          pltpu.VMEM((CH, H), jnp.float32),
          pltpu.VMEM((CH, H), jnp.float32),
          pltpu.SemaphoreType.DMA,
          pltpu.SemaphoreType.DMA,
      ],
  )
  return k(a, b, src3, dst3)


def _scatter_body(m_hbm, dst_hbm, z_hbm, p_hbm, dstv, rowbuf, acc):
  c = lax.axis_index("c")
  s = lax.axis_index("s")
  w = s * NC + c
  pltpu.sync_copy(dst_hbm.at[w], dstv)
  pltpu.sync_copy(z_hbm.at[pl.ds(s * NPS, NPS)], acc.at[pl.ds(s * NPS, NPS)])

  @pl.when(s == 0)
  def _():
    pltpu.sync_copy(z_hbm.at[pl.ds(NS * NPS, NTL)], acc.at[pl.ds(NS * NPS, NTL)])

  plsc.subcore_barrier()

  def chunk(j, _):
    pltpu.sync_copy(m_hbm.at[pl.ds(w * EPW + j * CH, CH)], rowbuf)
    pltpu.sync_copy(rowbuf, acc.at[dstv.at[j]], add=True)
    return 0

  lax.fori_loop(0, NCH, chunk, 0)
  plsc.subcore_barrier()
  pltpu.sync_copy(acc.at[pl.ds(s * NPS, NPS)], p_hbm.at[c, pl.ds(s * NPS, NPS)])

  @pl.when(s == 0)
  def _():
    pltpu.sync_copy(acc.at[pl.ds(NS * NPS, NTL)],
                    p_hbm.at[c, pl.ds(NS * NPS, NTL)])


def _sc_scatter(m, dst3, zeros):
  k = pl.kernel(
      _scatter_body,
      out_type=jax.ShapeDtypeStruct((NC, N, H), jnp.float32),
      mesh=_vmesh(),
      scratch_types=[
          pltpu.VMEM((NCH, CH), jnp.int32),
          pltpu.VMEM((CH, H), jnp.float32),
          pltpu.VMEM_SHARED((N, H), jnp.float32),
      ],
  )
  return k(m, dst3, zeros)


def _edge_mlp(pre_extra, ed, e_w, e_b, w1c, w2, b2, sw, sb):
  h = _silu(ed * e_w + e_b)
  c = jnp.dot(h, w1c, preferred_element_type=jnp.float32)
  m = _silu(c + pre_extra)
  m = _silu(jnp.dot(m, w2, preferred_element_type=jnp.float32) + b2)
  g = _sigmoid(jnp.dot(m, sw, preferred_element_type=jnp.float32) + sb)
  return m * g


def _edge0_kernel(ed_ref, ne_ref, w1ab_ref, b1_ref, ew_ref, ebias_ref,
                  w1c_ref, w2_ref, b2_ref, sw_ref, sb_ref, out_ref):
  ab0 = jnp.dot(ne_ref[...], w1ab_ref[...],
                preferred_element_type=jnp.float32) + b1_ref[...]
  out_ref[...] = _edge_mlp(ab0, ed_ref[...], ew_ref[...], ebias_ref[...],
                           w1c_ref[...], w2_ref[...], b2_ref[...],
                           sw_ref[...], sb_ref[...])


def _edge_kernel(g_ref, ed_ref, ew_ref, ebias_ref,
                 w1c_ref, w2_ref, b2_ref, sw_ref, sb_ref, out_ref):
  out_ref[...] = _edge_mlp(g_ref[...], ed_ref[...], ew_ref[...], ebias_ref[...],
                           w1c_ref[...], w2_ref[...], b2_ref[...],
                           sw_ref[...], sb_ref[...])


def _row_spec(bs):
  return pl.BlockSpec(bs, lambda i: (i, 0))


def _const_spec(shape):
  nd = len(shape)
  if nd == 2:
    return pl.BlockSpec(shape, lambda i: (0, 0))
  return pl.BlockSpec(shape, lambda i: (0, 0, 0))


def _edge_call_l0(ed, ne, w1ab, b1, ew, ebias, w1c, w2, b2, sw, sb):
  return pl.pallas_call(
      _edge0_kernel,
      grid=(E // EB,),
      in_specs=[
          _row_spec((EB, 1)),
          _const_spec((1, H)), _const_spec((H, H)), _const_spec((1, H)),
          _const_spec((1, H)), _const_spec((1, H)),
          _const_spec((H, H)), _const_spec((H, H)), _const_spec((1, H)),
          _const_spec((H, 1)), _const_spec((1, 1)),
      ],
      out_specs=_row_spec((EB, H)),
      out_shape=jax.ShapeDtypeStruct((E, H), jnp.float32),
  )(ed, ne, w1ab, b1, ew, ebias, w1c, w2, b2, sw, sb)


def _edge_call(g, ed, ew, ebias, w1c, w2, b2, sw, sb):
  return pl.pallas_call(
      _edge_kernel,
      grid=(E // EB,),
      in_specs=[
          _row_spec((EB, H)), _row_spec((EB, 1)),
          _const_spec((1, H)), _const_spec((1, H)),
          _const_spec((H, H)), _const_spec((H, H)), _const_spec((1, H)),
          _const_spec((H, 1)), _const_spec((1, 1)),
      ],
      out_specs=_row_spec((EB, H)),
      out_shape=jax.ShapeDtypeStruct((E, H), jnp.float32),
  )(g, ed, ew, ebias, w1c, w2, b2, sw, sb)


def _update_common(f, p0, p1, uw1, ub1, uw2, ub2):
  t = p0 + p1 + f
  hn = _silu(jnp.dot(t, uw1, preferred_element_type=jnp.float32) + ub1)
  hn = jnp.dot(hn, uw2, preferred_element_type=jnp.float32) + ub2
  return f + hn


def _update_kernel(f_ref, p_ref, uw1_ref, ub1_ref, uw2_ref, ub2_ref,
                   w1a_ref, w1b_ref, b1_ref, fn_ref, a_ref, b_ref):
  fn = _update_common(f_ref[...], p_ref[0], p_ref[1], uw1_ref[...],
                      ub1_ref[...], uw2_ref[...], ub2_ref[...])
  fn_ref[...] = fn
  a_ref[...] = jnp.dot(fn, w1a_ref[...],
                       preferred_element_type=jnp.float32) + b1_ref[...]
  b_ref[...] = jnp.dot(fn, w1b_ref[...], preferred_element_type=jnp.float32)


def _update_call(f, p, uw1, ub1, uw2, ub2, w1a, w1b, b1):
  return pl.pallas_call(
      _update_kernel,
      grid=(N // NB,),
      in_specs=[
          _row_spec((NB, H)),
          pl.BlockSpec((NC, NB, H), lambda i: (0, i, 0)),
          _const_spec((H, H)), _const_spec((1, H)),
          _const_spec((H, H)), _const_spec((1, H)),
          _const_spec((H, H)), _const_spec((H, H)), _const_spec((1, H)),
      ],
      out_specs=[_row_spec((NB, H)), _row_spec((NB, H)), _row_spec((NB, H))],
      out_shape=[jax.ShapeDtypeStruct((N, H), jnp.float32)] * 3,
  )(f, p, uw1, ub1, uw2, ub2, w1a, w1b, b1)


def _final_kernel(f_ref, p_ref, uw1_ref, ub1_ref, uw2_ref, ub2_ref,
                  o1_ref, o1b_ref, o2_ref, o2b_ref, sum_ref):
  fn = _update_common(f_ref[...], p_ref[0], p_ref[1], uw1_ref[...],
                      ub1_ref[...], uw2_ref[...], ub2_ref[...])
  g = _silu(jnp.dot(fn, o1_ref[...],
                    preferred_element_type=jnp.float32) + o1b_ref[...])
  g = jnp.dot(g, o2_ref[...], preferred_element_type=jnp.float32) + o2b_ref[...]
  bs = jnp.sum(g, axis=0, keepdims=True)

  @pl.when(pl.program_id(0) == 0)
  def _():
    sum_ref[...] = jnp.zeros_like(sum_ref)

  sum_ref[...] += bs


def _final_call(f, p, uw1, ub1, uw2, ub2, o1, o1b, o2, o2b):
  return pl.pallas_call(
      _final_kernel,
      grid=(N // NB,),
      in_specs=[
          _row_spec((NB, H)),
          pl.BlockSpec((NC, NB, H), lambda i: (0, i, 0)),
          _const_spec((H, H)), _const_spec((1, H)),
          _const_spec((H, H)), _const_spec((1, H)),
          _const_spec((H, H)), _const_spec((1, H)),
          _const_spec((H, H)), _const_spec((1, H)),
      ],
      out_specs=_const_spec((1, H)),
      out_shape=jax.ShapeDtypeStruct((1, H), jnp.float32),
  )(f, p, uw1, ub1, uw2, ub2, o1, o1b, o2, o2b)


def _readout_kernel(s_ref, ro1_ref, ro1b_ref, ro2_ref, ro2b_ref, out_ref):
  s = s_ref[...]
  r = jnp.concatenate([s, s * (1.0 / N)], axis=1)
  h = _silu(jnp.dot(r, ro1_ref[...],
                    preferred_element_type=jnp.float32) + ro1b_ref[...])
  out_ref[...] = jnp.dot(h, ro2_ref[...],
                         preferred_element_type=jnp.float32) + ro2b_ref[...]


def _readout_call(s, ro1, ro1b, ro2, ro2b):
  return pl.pallas_call(
      _readout_kernel,
      out_shape=jax.ShapeDtypeStruct((1, TARGET), jnp.float32),
  )(s, ro1, ro1b, ro2, ro2b)


def kernel(edge_dist, node_emb, edge_W, edge_b, msg_W1, msg_b1, msg_W2, msg_b2,
           soft_W, soft_b, upd_W1, upd_b1, upd_W2, upd_b2,
           out1_W, out1_b, out2_W, out2_b, ro1_W, ro1_b, ro2_W, ro2_b,
           edge_index):
  src3 = edge_index[0].reshape(NW, NCH, CH)
  dst3 = edge_index[1].reshape(NW, NCH, CH)
  zeros = jnp.zeros((N, H), jnp.float32)
  ne = node_emb.reshape(1, H)
  ew = edge_W.reshape(1, H)
  ebias = edge_b.reshape(1, H)

  def r2(x):
    return x.reshape(1, -1)

  w1ab0 = msg_W1[0, :H] + msg_W1[0, H:2 * H]
  m = _edge_call_l0(edge_dist, ne, w1ab0, r2(msg_b1[0]), ew, ebias,
                    msg_W1[0, 2 * H:], msg_W2[0], r2(msg_b2[0]),
                    soft_W[0], r2(soft_b[0]))
  p = _sc_scatter(m, dst3, zeros)
  feat = jnp.broadcast_to(node_emb[None, :], (N, H))
  feat, a, b = _update_call(feat, p, upd_W1[0], r2(upd_b1[0]),
                            upd_W2[0], r2(upd_b2[0]),
                            msg_W1[1, :H], msg_W1[1, H:2 * H], r2(msg_b1[1]))

  for l in range(1, L):
    g = _sc_gather(a, b, src3, dst3)
    m = _edge_call(g, edge_dist, ew, ebias, msg_W1[l, 2 * H:],
                   msg_W2[l], r2(msg_b2[l]), soft_W[l], r2(soft_b[l]))
    p = _sc_scatter(m, dst3, zeros)
    if l < L - 1:
      feat, a, b = _update_call(feat, p, upd_W1[l], r2(upd_b1[l]),
                                upd_W2[l], r2(upd_b2[l]),
                                msg_W1[l + 1, :H], msg_W1[l + 1, H:2 * H],
                                r2(msg_b1[l + 1]))
    else:
      sums = _final_call(feat, p, upd_W1[l], r2(upd_b1[l]),
                         upd_W2[l], r2(upd_b2[l]),
                         out1_W, r2(out1_b), out2_W, r2(out2_b))

  return _readout_call(sums, ro1_W, r2(ro1_b), ro2_W, r2(ro2_b))


# speedup vs baseline: 4.9642x; 1.3619x over previous
import functools

import jax
import jax.numpy as jnp
from jax import lax
from jax.experimental import pallas as pl
from jax.experimental.pallas import tpu as pltpu
from jax.experimental.pallas import tpu_sc as plsc

N = 10000
E = 320000
H = 128
L = 4
TARGET = 32

NC = 2
NS = 16
NW = NC * NS
EPW = E // NW
CH = 80
NCH = EPW // CH
CHS = 40
NCHS = EPW // CHS
NSLOTS = 3
NPS = 624
NTL = N - NS * NPS

EB = 4000
NB = 2000

_silu = jax.nn.silu
_sigmoid = jax.nn.sigmoid

def _vmesh():
  return plsc.VectorSubcoreMesh(core_axis_name="c", subcore_axis_name="s",
                                num_cores=NC, num_subcores=NS)


NSLOT = 3


def _gather_body(a_hbm, b_hbm, src_hbm, dst_hbm, g_hbm,
                 srcv, dstv, buf_a, buf_b, sem_a, sem_b, sem_o):
  c = lax.axis_index("c")
  s = lax.axis_index("s")
  w = s * NC + c
  pltpu.sync_copy(src_hbm.at[w], srcv)
  pltpu.sync_copy(dst_hbm.at[w], dstv)

  def start_in(j, t):
    pltpu.async_copy(a_hbm.at[srcv.at[j]], buf_a.at[t], sem_a.at[t])
    pltpu.async_copy(b_hbm.at[dstv.at[j]], buf_b.at[t], sem_b.at[t])

  def wait_in(j, t):
    pltpu.make_async_copy(a_hbm.at[srcv.at[j]], buf_a.at[t], sem_a.at[t]).wait()
    pltpu.make_async_copy(b_hbm.at[dstv.at[j]], buf_b.at[t], sem_b.at[t]).wait()

  def out_desc(j, t):
    return pltpu.make_async_copy(
        buf_a.at[t], g_hbm.at[pl.ds(w * EPW + j * CH, CH)], sem_o.at[t])

  for t in range(NSLOT - 1):
    start_in(t, t)

  def add_chunk(j, t):
    wait_in(j, t)

    def add_row(r, _):
      for q in range(H // 16):
        x = buf_b[t, r, pl.ds(q * 16, 16)]
        plsc.addupdate(buf_a.at[t, r, pl.ds(q * 16, 16)], x)
      return 0

    lax.fori_loop(0, CH, add_row, 0)
    out_desc(j, t).start()

  def outer(jb, _):
    for t in range(NSLOT):
      j = jb * NSLOT + t
      add_chunk(j, t)
      tn = (t + NSLOT - 1) % NSLOT
      if t == 0:
        @pl.when(j >= 1)
        def _():
          out_desc(j - 1, tn).wait()

        @pl.when(j + NSLOT - 1 < NCH)
        def _():
          start_in(j + NSLOT - 1, tn)
      else:

        @pl.when(j + NSLOT - 1 < NCH)
        def _():
          out_desc(j - 1, tn).wait()
          start_in(j + NSLOT - 1, tn)
    return 0

  nfull = NCH // NSLOT
  lax.fori_loop(0, nfull, outer, 0)
  for j in range(nfull * NSLOT, NCH):
    add_chunk(j, j % NSLOT)
  for j in range(NCH - NSLOT, NCH):
    out_desc(j, j % NSLOT).wait()


def _sc_gather(a, b, src3, dst3):
  k = pl.kernel(
      _gather_body,
      out_type=jax.ShapeDtypeStruct((E, H), jnp.float32),
      mesh=_vmesh(),
      scratch_types=[
          pltpu.VMEM((NCH, CH), jnp.int32),
          pltpu.VMEM((NCH, CH), jnp.int32),
          pltpu.VMEM((NSLOT, CH, H), jnp.float32),
          pltpu.VMEM((NSLOT, CH, H), jnp.float32),
          pltpu.SemaphoreType.DMA((NSLOT,)),
          pltpu.SemaphoreType.DMA((NSLOT,)),
          pltpu.SemaphoreType.DMA((NSLOT,)),
      ],
  )
  return k(a, b, src3, dst3)


def _scatter_body(m_hbm, dst_hbm, z_hbm, p_hbm, dstv, rowbuf, acc,
                  sem_i, sem_d):
  c = lax.axis_index("c")
  s = lax.axis_index("s")
  w = s * NC + c
  pltpu.sync_copy(dst_hbm.at[w], dstv)
  pltpu.sync_copy(z_hbm.at[pl.ds(s * NPS, NPS)], acc.at[pl.ds(s * NPS, NPS)])

  @pl.when(s == 0)
  def _():
    pltpu.sync_copy(z_hbm.at[pl.ds(NS * NPS, NTL)], acc.at[pl.ds(NS * NPS, NTL)])

  plsc.subcore_barrier()

  def in_desc(j, t):
    return pltpu.make_async_copy(
        m_hbm.at[pl.ds(w * EPW + j * CHS, CHS)], rowbuf.at[t], sem_i.at[t])

  def add_desc(j, t):
    return pltpu.make_async_copy(rowbuf.at[t], acc.at[dstv.at[j]], sem_d.at[t])

  for t in range(NSLOTS - 1):
    in_desc(t, t).start()

  def visit(j, t):
    in_desc(j, t).wait()
    pltpu.async_copy(rowbuf.at[t], acc.at[dstv.at[j]], sem_d.at[t], add=True)
    tn = (t + NSLOTS - 1) % NSLOTS
    if t == 0:
      @pl.when(j >= 1)
      def _():
        add_desc(j - 1, tn).wait()

      @pl.when(j + NSLOTS - 1 < NCHS)
      def _():
        in_desc(j + NSLOTS - 1, tn).start()
    else:

      @pl.when(j + NSLOTS - 1 < NCHS)
      def _():
        add_desc(j - 1, tn).wait()
        in_desc(j + NSLOTS - 1, tn).start()

  def outer(jb, _):
    for t in range(NSLOTS):
      visit(jb * NSLOTS + t, t)
    return 0

  nfull = NCHS // NSLOTS
  lax.fori_loop(0, nfull, outer, 0)
  for j in range(nfull * NSLOTS, NCHS):
    in_desc(j, j % NSLOTS).wait()
    pltpu.async_copy(rowbuf.at[j % NSLOTS], acc.at[dstv.at[j]],
                     sem_d.at[j % NSLOTS], add=True)
  for t in range(NSLOTS):
    j = NCHS - NSLOTS + t
    add_desc(j, j % NSLOTS).wait()
  plsc.subcore_barrier()
  pltpu.sync_copy(acc.at[pl.ds(s * NPS, NPS)], p_hbm.at[c, pl.ds(s * NPS, NPS)])

  @pl.when(s == 0)
  def _():
    pltpu.sync_copy(acc.at[pl.ds(NS * NPS, NTL)],
                    p_hbm.at[c, pl.ds(NS * NPS, NTL)])


def _sc_scatter(m, dst3s, zeros):
  k = pl.kernel(
      _scatter_body,
      out_type=jax.ShapeDtypeStruct((NC, N, H), jnp.float32),
      mesh=_vmesh(),
      scratch_types=[
          pltpu.VMEM((NCHS, CHS), jnp.int32),
          pltpu.VMEM((NSLOTS, CHS, H), jnp.float32),
          pltpu.VMEM_SHARED((N, H), jnp.float32),
          pltpu.SemaphoreType.DMA((NSLOTS,)),
          pltpu.SemaphoreType.DMA((NSLOTS,)),
      ],
  )
  return k(m, dst3s, zeros)


def _edge_mlp(pre_extra, ed, e_w, e_b, w1c, w2, b2, sw, sb):
  h = _silu(ed * e_w + e_b)
  c = jnp.dot(h, w1c, preferred_element_type=jnp.float32)
  m = _silu(c + pre_extra)
  m = _silu(jnp.dot(m, w2, preferred_element_type=jnp.float32) + b2)
  g = _sigmoid(jnp.dot(m, sw, preferred_element_type=jnp.float32) + sb)
  return m * g


def _edge0_kernel(ed_ref, ne_ref, w1ab_ref, b1_ref, ew_ref, ebias_ref,
                  w1c_ref, w2_ref, b2_ref, sw_ref, sb_ref, out_ref):
  ab0 = jnp.dot(ne_ref[...], w1ab_ref[...],
                preferred_element_type=jnp.float32) + b1_ref[...]
  out_ref[...] = _edge_mlp(ab0, ed_ref[...], ew_ref[...], ebias_ref[...],
                           w1c_ref[...], w2_ref[...], b2_ref[...],
                           sw_ref[...], sb_ref[...])


def _edge_kernel(g_ref, ed_ref, ew_ref, ebias_ref,
                 w1c_ref, w2_ref, b2_ref, sw_ref, sb_ref, out_ref):
  out_ref[...] = _edge_mlp(g_ref[...], ed_ref[...], ew_ref[...], ebias_ref[...],
                           w1c_ref[...], w2_ref[...], b2_ref[...],
                           sw_ref[...], sb_ref[...])


def _row_spec(bs):
  return pl.BlockSpec(bs, lambda i: (i, 0))


def _const_spec(shape):
  nd = len(shape)
  if nd == 2:
    return pl.BlockSpec(shape, lambda i: (0, 0))
  return pl.BlockSpec(shape, lambda i: (0, 0, 0))


def _edge_call_l0(ed, ne, w1ab, b1, ew, ebias, w1c, w2, b2, sw, sb):
  return pl.pallas_call(
      _edge0_kernel,
      grid=(E // EB,),
      in_specs=[
          _row_spec((EB, 1)),
          _const_spec((1, H)), _const_spec((H, H)), _const_spec((1, H)),
          _const_spec((1, H)), _const_spec((1, H)),
          _const_spec((H, H)), _const_spec((H, H)), _const_spec((1, H)),
          _const_spec((H, 1)), _const_spec((1, 1)),
      ],
      out_specs=_row_spec((EB, H)),
      out_shape=jax.ShapeDtypeStruct((E, H), jnp.float32),
  )(ed, ne, w1ab, b1, ew, ebias, w1c, w2, b2, sw, sb)


def _edge_call(g, ed, ew, ebias, w1c, w2, b2, sw, sb):
  return pl.pallas_call(
      _edge_kernel,
      grid=(E // EB,),
      in_specs=[
          _row_spec((EB, H)), _row_spec((EB, 1)),
          _const_spec((1, H)), _const_spec((1, H)),
          _const_spec((H, H)), _const_spec((H, H)), _const_spec((1, H)),
          _const_spec((H, 1)), _const_spec((1, 1)),
      ],
      out_specs=_row_spec((EB, H)),
      out_shape=jax.ShapeDtypeStruct((E, H), jnp.float32),
  )(g, ed, ew, ebias, w1c, w2, b2, sw, sb)


def _update_common(f, p0, p1, uw1, ub1, uw2, ub2):
  t = p0 + p1 + f
  hn = _silu(jnp.dot(t, uw1, preferred_element_type=jnp.float32) + ub1)
  hn = jnp.dot(hn, uw2, preferred_element_type=jnp.float32) + ub2
  return f + hn


def _update_kernel(f_ref, p_ref, uw1_ref, ub1_ref, uw2_ref, ub2_ref,
                   w1a_ref, w1b_ref, b1_ref, fn_ref, a_ref, b_ref):
  fn = _update_common(f_ref[...], p_ref[0], p_ref[1], uw1_ref[...],
                      ub1_ref[...], uw2_ref[...], ub2_ref[...])
  fn_ref[...] = fn
  a_ref[...] = jnp.dot(fn, w1a_ref[...],
                       preferred_element_type=jnp.float32) + b1_ref[...]
  b_ref[...] = jnp.dot(fn, w1b_ref[...], preferred_element_type=jnp.float32)


def _update_call(f, p, uw1, ub1, uw2, ub2, w1a, w1b, b1):
  return pl.pallas_call(
      _update_kernel,
      grid=(N // NB,),
      in_specs=[
          _row_spec((NB, H)),
          pl.BlockSpec((NC, NB, H), lambda i: (0, i, 0)),
          _const_spec((H, H)), _const_spec((1, H)),
          _const_spec((H, H)), _const_spec((1, H)),
          _const_spec((H, H)), _const_spec((H, H)), _const_spec((1, H)),
      ],
      out_specs=[_row_spec((NB, H)), _row_spec((NB, H)), _row_spec((NB, H))],
      out_shape=[jax.ShapeDtypeStruct((N, H), jnp.float32)] * 3,
  )(f, p, uw1, ub1, uw2, ub2, w1a, w1b, b1)


def _final_kernel(f_ref, p_ref, uw1_ref, ub1_ref, uw2_ref, ub2_ref,
                  o1_ref, o1b_ref, o2_ref, o2b_ref, sum_ref):
  fn = _update_common(f_ref[...], p_ref[0], p_ref[1], uw1_ref[...],
                      ub1_ref[...], uw2_ref[...], ub2_ref[...])
  g = _silu(jnp.dot(fn, o1_ref[...],
                    preferred_element_type=jnp.float32) + o1b_ref[...])
  g = jnp.dot(g, o2_ref[...], preferred_element_type=jnp.float32) + o2b_ref[...]
  bs = jnp.sum(g, axis=0, keepdims=True)

  @pl.when(pl.program_id(0) == 0)
  def _():
    sum_ref[...] = jnp.zeros_like(sum_ref)

  sum_ref[...] += bs


def _final_call(f, p, uw1, ub1, uw2, ub2, o1, o1b, o2, o2b):
  return pl.pallas_call(
      _final_kernel,
      grid=(N // NB,),
      in_specs=[
          _row_spec((NB, H)),
          pl.BlockSpec((NC, NB, H), lambda i: (0, i, 0)),
          _const_spec((H, H)), _const_spec((1, H)),
          _const_spec((H, H)), _const_spec((1, H)),
          _const_spec((H, H)), _const_spec((1, H)),
          _const_spec((H, H)), _const_spec((1, H)),
      ],
      out_specs=_const_spec((1, H)),
      out_shape=jax.ShapeDtypeStruct((1, H), jnp.float32),
  )(f, p, uw1, ub1, uw2, ub2, o1, o1b, o2, o2b)


def _readout_kernel(s_ref, ro1_ref, ro1b_ref, ro2_ref, ro2b_ref, out_ref):
  s = s_ref[...]
  r = jnp.concatenate([s, s * (1.0 / N)], axis=1)
  h = _silu(jnp.dot(r, ro1_ref[...],
                    preferred_element_type=jnp.float32) + ro1b_ref[...])
  out_ref[...] = jnp.dot(h, ro2_ref[...],
                         preferred_element_type=jnp.float32) + ro2b_ref[...]


def _readout_call(s, ro1, ro1b, ro2, ro2b):
  return pl.pallas_call(
      _readout_kernel,
      out_shape=jax.ShapeDtypeStruct((1, TARGET), jnp.float32),
  )(s, ro1, ro1b, ro2, ro2b)


def kernel(edge_dist, node_emb, edge_W, edge_b, msg_W1, msg_b1, msg_W2, msg_b2,
           soft_W, soft_b, upd_W1, upd_b1, upd_W2, upd_b2,
           out1_W, out1_b, out2_W, out2_b, ro1_W, ro1_b, ro2_W, ro2_b,
           edge_index):
  src3 = edge_index[0].reshape(NW, NCH, CH)
  dst3 = edge_index[1].reshape(NW, NCH, CH)
  dst3s = edge_index[1].reshape(NW, NCHS, CHS)
  zeros = jnp.zeros((N, H), jnp.float32)
  ne = node_emb.reshape(1, H)
  ew = edge_W.reshape(1, H)
  ebias = edge_b.reshape(1, H)

  def r2(x):
    return x.reshape(1, -1)

  w1ab0 = msg_W1[0, :H] + msg_W1[0, H:2 * H]
  m = _edge_call_l0(edge_dist, ne, w1ab0, r2(msg_b1[0]), ew, ebias,
                    msg_W1[0, 2 * H:], msg_W2[0], r2(msg_b2[0]),
                    soft_W[0], r2(soft_b[0]))
  p = _sc_scatter(m, dst3s, zeros)
  feat = jnp.broadcast_to(node_emb[None, :], (N, H))
  feat, a, b = _update_call(feat, p, upd_W1[0], r2(upd_b1[0]),
                            upd_W2[0], r2(upd_b2[0]),
                            msg_W1[1, :H], msg_W1[1, H:2 * H], r2(msg_b1[1]))

  for l in range(1, L):
    g = _sc_gather(a, b, src3, dst3)
    m = _edge_call(g, edge_dist, ew, ebias, msg_W1[l, 2 * H:],
                   msg_W2[l], r2(msg_b2[l]), soft_W[l], r2(soft_b[l]))
    p = _sc_scatter(m, dst3s, zeros)
    if l < L - 1:
      feat, a, b = _update_call(feat, p, upd_W1[l], r2(upd_b1[l]),
                                upd_W2[l], r2(upd_b2[l]),
                                msg_W1[l + 1, :H], msg_W1[l + 1, H:2 * H],
                                r2(msg_b1[l + 1]))
    else:
      sums = _final_call(feat, p, upd_W1[l], r2(upd_b1[l]),
                         upd_W2[l], r2(upd_b2[l]),
                         out1_W, r2(out1_b), out2_W, r2(out2_b))

  return _readout_call(sums, ro1_W, r2(ro1_b), ro2_W, r2(ro2_b))


# speedup vs baseline: 5.6172x; 1.1315x over previous
import jax
import jax.numpy as jnp
from jax import lax
from jax.experimental import pallas as pl
from jax.experimental.pallas import tpu as pltpu
from jax.experimental.pallas import tpu_sc as plsc

N = 10000
E = 320000
H = 128
L = 4
TARGET = 32

NC = 2
NS = 16
NW = NC * NS
EH0 = 161280
EHS = (EH0, E - EH0)
CH = 80
CHS = 40
NSLOT = 3
NSLOTS = 3
NPS = 624
NTL = N - NS * NPS

NB = 2000

_silu = jax.nn.silu
_sigmoid = jax.nn.sigmoid


def _vmesh():
  return plsc.VectorSubcoreMesh(core_axis_name="c", subcore_axis_name="s",
                                num_cores=NC, num_subcores=NS)


def _sc_gather(a, b, src3, dst3, eh):
  epwh = eh // NW
  nch = epwh // CH

  def body(a_hbm, b_hbm, src_hbm, dst_hbm, g_hbm,
           srcv, dstv, buf_a, buf_b, sem_a, sem_b, sem_o):
    c = lax.axis_index("c")
    s = lax.axis_index("s")
    w = s * NC + c
    pltpu.sync_copy(src_hbm.at[w], srcv)
    pltpu.sync_copy(dst_hbm.at[w], dstv)

    def start_in(j, t):
      pltpu.async_copy(a_hbm.at[srcv.at[j]], buf_a.at[t], sem_a.at[t])
      pltpu.async_copy(b_hbm.at[dstv.at[j]], buf_b.at[t], sem_b.at[t])

    def wait_in(j, t):
      pltpu.make_async_copy(a_hbm.at[srcv.at[j]], buf_a.at[t],
                            sem_a.at[t]).wait()
      pltpu.make_async_copy(b_hbm.at[dstv.at[j]], buf_b.at[t],
                            sem_b.at[t]).wait()

    def out_desc(j, t):
      return pltpu.make_async_copy(
          buf_a.at[t], g_hbm.at[pl.ds(w * epwh + j * CH, CH)], sem_o.at[t])

    for t in range(NSLOT - 1):
      start_in(t, t)

    def add_chunk(j, t):
      wait_in(j, t)

      @plsc.parallel_loop(0, CH * (H // 16), unroll=8)
      def _(i):
        r = i >> 3
        q = (i & 7) * 16
        x = buf_b[t, r, pl.ds(q, 16)]
        plsc.addupdate(buf_a.at[t, r, pl.ds(q, 16)], x)

      out_desc(j, t).start()

    def outer(jb, _):
      for t in range(NSLOT):
        j = jb * NSLOT + t
        add_chunk(j, t)
        tn = (t + NSLOT - 1) % NSLOT
        if t == 0:
          @pl.when(j >= 1)
          def _():
            out_desc(j - 1, tn).wait()

          @pl.when(j + NSLOT - 1 < nch)
          def _():
            start_in(j + NSLOT - 1, tn)
        else:

          @pl.when(j + NSLOT - 1 < nch)
          def _():
            out_desc(j - 1, tn).wait()
            start_in(j + NSLOT - 1, tn)
      return 0

    nfull = nch // NSLOT
    lax.fori_loop(0, nfull, outer, 0)
    for j in range(nfull * NSLOT, nch):
      add_chunk(j, j % NSLOT)
    for j in range(nch - NSLOT, nch):
      out_desc(j, j % NSLOT).wait()

  k = pl.kernel(
      body,
      out_type=jax.ShapeDtypeStruct((eh, H), jnp.float32),
      mesh=_vmesh(),
      scratch_types=[
          pltpu.VMEM((nch, CH), jnp.int32),
          pltpu.VMEM((nch, CH), jnp.int32),
          pltpu.VMEM((NSLOT, CH, H), jnp.float32),
          pltpu.VMEM((NSLOT, CH, H), jnp.float32),
          pltpu.SemaphoreType.DMA((NSLOT,)),
          pltpu.SemaphoreType.DMA((NSLOT,)),
          pltpu.SemaphoreType.DMA((NSLOT,)),
      ],
  )
  return k(a, b, src3, dst3)


def _sc_scatter(m, dst3s, init, eh):
  epwh = eh // NW
  nchs = epwh // CHS

  def body(m_hbm, dst_hbm, z_hbm, p_hbm, dstv, rowbuf, acc, sem_i, sem_d):
    c = lax.axis_index("c")
    s = lax.axis_index("s")
    w = s * NC + c
    pltpu.sync_copy(dst_hbm.at[w], dstv)
    pltpu.sync_copy(z_hbm.at[c, pl.ds(s * NPS, NPS)],
                    acc.at[pl.ds(s * NPS, NPS)])

    @pl.when(s == 0)
    def _():
      pltpu.sync_copy(z_hbm.at[c, pl.ds(NS * NPS, NTL)],
                      acc.at[pl.ds(NS * NPS, NTL)])

    plsc.subcore_barrier()

    def in_desc(j, t):
      return pltpu.make_async_copy(
          m_hbm.at[pl.ds(w * epwh + j * CHS, CHS)], rowbuf.at[t], sem_i.at[t])

    def add_desc(j, t):
      return pltpu.make_async_copy(rowbuf.at[t], acc.at[dstv.at[j]],
                                   sem_d.at[t])

    for t in range(NSLOTS - 1):
      in_desc(t, t).start()

    def visit(j, t):
      in_desc(j, t).wait()
      pltpu.async_copy(rowbuf.at[t], acc.at[dstv.at[j]], sem_d.at[t],
                       add=True)
      tn = (t + NSLOTS - 1) % NSLOTS
      if t == 0:
        @pl.when(j >= 1)
        def _():
          add_desc(j - 1, tn).wait()

        @pl.when(j + NSLOTS - 1 < nchs)
        def _():
          in_desc(j + NSLOTS - 1, tn).start()
      else:

        @pl.when(j + NSLOTS - 1 < nchs)
        def _():
          add_desc(j - 1, tn).wait()
          in_desc(j + NSLOTS - 1, tn).start()

    def outer(jb, _):
      for t in range(NSLOTS):
        visit(jb * NSLOTS + t, t)
      return 0

    nfull = nchs // NSLOTS
    lax.fori_loop(0, nfull, outer, 0)
    for j in range(nfull * NSLOTS, nchs):
      in_desc(j, j % NSLOTS).wait()
      pltpu.async_copy(rowbuf.at[j % NSLOTS], acc.at[dstv.at[j]],
                       sem_d.at[j % NSLOTS], add=True)
    for t in range(NSLOTS):
      j = nchs - NSLOTS + t
      add_desc(j, j % NSLOTS).wait()
    plsc.subcore_barrier()
    pltpu.sync_copy(acc.at[pl.ds(s * NPS, NPS)],
                    p_hbm.at[c, pl.ds(s * NPS, NPS)])

    @pl.when(s == 0)
    def _():
      pltpu.sync_copy(acc.at[pl.ds(NS * NPS, NTL)],
                      p_hbm.at[c, pl.ds(NS * NPS, NTL)])

  k = pl.kernel(
      body,
      out_type=jax.ShapeDtypeStruct((NC, N, H), jnp.float32),
      mesh=_vmesh(),
      scratch_types=[
          pltpu.VMEM((nchs, CHS), jnp.int32),
          pltpu.VMEM((NSLOTS, CHS, H), jnp.float32),
          pltpu.VMEM_SHARED((N, H), jnp.float32),
          pltpu.SemaphoreType.DMA((NSLOTS,)),
          pltpu.SemaphoreType.DMA((NSLOTS,)),
      ],
  )
  return k(m, dst3s, init)


def _edge_mlp(pre_extra, ed, e_w, e_b, w1c, w2, b2, sw, sb):
  h = _silu(ed * e_w + e_b)
  c = jnp.dot(h, w1c, preferred_element_type=jnp.float32)
  m = _silu(c + pre_extra)
  m = _silu(jnp.dot(m, w2, preferred_element_type=jnp.float32) + b2)
  g = _sigmoid(jnp.dot(m, sw, preferred_element_type=jnp.float32) + sb)
  return m * g


def _edge0_kernel(ed_ref, ne_ref, w1a_ref, w1b_ref, b1_ref, ew_ref, ebias_ref,
                  w1c_ref, w2_ref, b2_ref, sw_ref, sb_ref, out_ref):
  ab0 = (jnp.dot(ne_ref[...], w1a_ref[...], preferred_element_type=jnp.float32)
         + jnp.dot(ne_ref[...], w1b_ref[...],
                   preferred_element_type=jnp.float32) + b1_ref[...])
  out_ref[...] = _edge_mlp(ab0, ed_ref[...], ew_ref[...], ebias_ref[...],
                           w1c_ref[...], w2_ref[...], b2_ref[...],
                           sw_ref[...], sb_ref[...])


def _edge_kernel(g_ref, ed_ref, ew_ref, ebias_ref,
                 w1c_ref, w2_ref, b2_ref, sw_ref, sb_ref, out_ref):
  out_ref[...] = _edge_mlp(g_ref[...], ed_ref[...], ew_ref[...],
                           ebias_ref[...], w1c_ref[...], w2_ref[...],
                           b2_ref[...], sw_ref[...], sb_ref[...])


def _row_spec(bs):
  return pl.BlockSpec(bs, lambda i: (i, 0))


def _const_spec(shape):
  return pl.BlockSpec(shape, lambda i: (0,) * len(shape))


def _edge_call_l0(ed, ne, w1a, w1b, b1, ew, ebias, w1c, w2, b2, sw, sb):
  eh = ed.shape[0]
  eb = eh // 40
  return pl.pallas_call(
      _edge0_kernel,
      grid=(eh // eb,),
      in_specs=[
          _row_spec((eb, 1)),
          _const_spec((1, H)), _const_spec((H, H)), _const_spec((H, H)),
          _const_spec((1, H)),
          _const_spec((1, H)), _const_spec((1, H)),
          _const_spec((H, H)), _const_spec((H, H)), _const_spec((1, H)),
          _const_spec((H, 1)), _const_spec((1, 1)),
      ],
      out_specs=_row_spec((eb, H)),
      out_shape=jax.ShapeDtypeStruct((eh, H), jnp.float32),
  )(ed, ne, w1a, w1b, b1, ew, ebias, w1c, w2, b2, sw, sb)


def _edge_call(g, ed, ew, ebias, w1c, w2, b2, sw, sb):
  eh = ed.shape[0]
  eb = eh // 40
  return pl.pallas_call(
      _edge_kernel,
      grid=(eh // eb,),
      in_specs=[
          _row_spec((eb, H)), _row_spec((eb, 1)),
          _const_spec((1, H)), _const_spec((1, H)),
          _const_spec((H, H)), _const_spec((H, H)), _const_spec((1, H)),
          _const_spec((H, 1)), _const_spec((1, 1)),
      ],
      out_specs=_row_spec((eb, H)),
      out_shape=jax.ShapeDtypeStruct((eh, H), jnp.float32),
  )(g, ed, ew, ebias, w1c, w2, b2, sw, sb)


def _update_common(f, p0, p1, uw1, ub1, uw2, ub2):
  t = p0 + p1 + f
  hn = _silu(jnp.dot(t, uw1, preferred_element_type=jnp.float32) + ub1)
  hn = jnp.dot(hn, uw2, preferred_element_type=jnp.float32) + ub2
  return f + hn


def _update_kernel(f_ref, p_ref, uw1_ref, ub1_ref, uw2_ref, ub2_ref,
                   w1a_ref, w1b_ref, b1_ref, fn_ref, a_ref, b_ref):
  fn = _update_common(f_ref[...], p_ref[0], p_ref[1], uw1_ref[...],
                      ub1_ref[...], uw2_ref[...], ub2_ref[...])
  fn_ref[...] = fn
  a_ref[...] = jnp.dot(fn, w1a_ref[...],
                       preferred_element_type=jnp.float32) + b1_ref[...]
  b_ref[...] = jnp.dot(fn, w1b_ref[...], preferred_element_type=jnp.float32)


def _update_call(f, p, uw1, ub1, uw2, ub2, w1a, w1b, b1):
  return pl.pallas_call(
      _update_kernel,
      grid=(N // NB,),
      in_specs=[
          _row_spec((NB, H)),
          pl.BlockSpec((NC, NB, H), lambda i: (0, i, 0)),
          _const_spec((H, H)), _const_spec((1, H)),
          _const_spec((H, H)), _const_spec((1, H)),
          _const_spec((H, H)), _const_spec((H, H)), _const_spec((1, H)),
      ],
      out_specs=[_row_spec((NB, H)), _row_spec((NB, H)), _row_spec((NB, H))],
      out_shape=[jax.ShapeDtypeStruct((N, H), jnp.float32)] * 3,
  )(f, p, uw1, ub1, uw2, ub2, w1a, w1b, b1)


def _final_kernel(f_ref, p_ref, uw1_ref, ub1_ref, uw2_ref, ub2_ref,
                  o1_ref, o1b_ref, o2_ref, o2b_ref, sum_ref):
  fn = _update_common(f_ref[...], p_ref[0], p_ref[1], uw1_ref[...],
                      ub1_ref[...], uw2_ref[...], ub2_ref[...])
  g = _silu(jnp.dot(fn, o1_ref[...],
                    preferred_element_type=jnp.float32) + o1b_ref[...])
  g = jnp.dot(g, o2_ref[...], preferred_element_type=jnp.float32) + o2b_ref[...]
  bs = jnp.sum(g, axis=0, keepdims=True)

  @pl.when(pl.program_id(0) == 0)
  def _():
    sum_ref[...] = jnp.zeros_like(sum_ref)

  sum_ref[...] += bs


def _final_call(f, p, uw1, ub1, uw2, ub2, o1, o1b, o2, o2b):
  return pl.pallas_call(
      _final_kernel,
      grid=(N // NB,),
      in_specs=[
          _row_spec((NB, H)),
          pl.BlockSpec((NC, NB, H), lambda i: (0, i, 0)),
          _const_spec((H, H)), _const_spec((1, H)),
          _const_spec((H, H)), _const_spec((1, H)),
          _const_spec((H, H)), _const_spec((1, H)),
          _const_spec((H, H)), _const_spec((1, H)),
      ],
      out_specs=_const_spec((1, H)),
      out_shape=jax.ShapeDtypeStruct((1, H), jnp.float32),
  )(f, p, uw1, ub1, uw2, ub2, o1, o1b, o2, o2b)


def _readout_kernel(s_ref, ro1_ref, ro1b_ref, ro2_ref, ro2b_ref, out_ref):
  s = s_ref[...]
  r = jnp.concatenate([s, s * (1.0 / N)], axis=1)
  h = _silu(jnp.dot(r, ro1_ref[...],
                    preferred_element_type=jnp.float32) + ro1b_ref[...])
  out_ref[...] = jnp.dot(h, ro2_ref[...],
                         preferred_element_type=jnp.float32) + ro2b_ref[...]


def _readout_call(s, ro1, ro1b, ro2, ro2b):
  return pl.pallas_call(
      _readout_kernel,
      out_shape=jax.ShapeDtypeStruct((1, TARGET), jnp.float32),
  )(s, ro1, ro1b, ro2, ro2b)


def kernel(edge_dist, node_emb, edge_W, edge_b, msg_W1, msg_b1, msg_W2, msg_b2,
           soft_W, soft_b, upd_W1, upd_b1, upd_W2, upd_b2,
           out1_W, out1_b, out2_W, out2_b, ro1_W, ro1_b, ro2_W, ro2_b,
           edge_index):
  src = edge_index[0]
  dst = edge_index[1]
  lo = [0, EHS[0], E]
  srcs = [src[lo[h]:lo[h + 1]].reshape(NW, EHS[h] // NW // CH, CH)
          for h in range(2)]
  dsts = [dst[lo[h]:lo[h + 1]].reshape(NW, EHS[h] // NW // CH, CH)
          for h in range(2)]
  dstss = [dst[lo[h]:lo[h + 1]].reshape(NW, EHS[h] // NW // CHS, CHS)
           for h in range(2)]
  eds = [edge_dist[lo[h]:lo[h + 1]] for h in range(2)]
  zeros = jnp.zeros((NC, N, H), jnp.float32)
  ne = node_emb.reshape(1, H)
  ew = edge_W.reshape(1, H)
  ebias = edge_b.reshape(1, H)

  def r2(x):
    return x.reshape(1, -1)

  def seg_sum(ms):
    p = zeros
    for h in range(2):
      p = _sc_scatter(ms[h], dstss[h], p, EHS[h])
    return p

  ms = [_edge_call_l0(eds[h], ne, msg_W1[0, :H], msg_W1[0, H:2 * H],
                      r2(msg_b1[0]), ew, ebias,
                      msg_W1[0, 2 * H:], msg_W2[0], r2(msg_b2[0]),
                      soft_W[0], r2(soft_b[0])) for h in range(2)]
  p = seg_sum(ms)
  feat = jnp.broadcast_to(node_emb[None, :], (N, H))
  feat, a, b = _update_call(feat, p, upd_W1[0], r2(upd_b1[0]),
                            upd_W2[0], r2(upd_b2[0]),
                            msg_W1[1, :H], msg_W1[1, H:2 * H], r2(msg_b1[1]))

  for l in range(1, L):
    gs = [_sc_gather(a, b, srcs[h], dsts[h], EHS[h]) for h in range(2)]
    ms = [_edge_call(gs[h], eds[h], ew, ebias, msg_W1[l, 2 * H:],
                     msg_W2[l], r2(msg_b2[l]), soft_W[l], r2(soft_b[l]))
          for h in range(2)]
    p = seg_sum(ms)
    if l < L - 1:
      feat, a, b = _update_call(feat, p, upd_W1[l], r2(upd_b1[l]),
                                upd_W2[l], r2(upd_b2[l]),
                                msg_W1[l + 1, :H], msg_W1[l + 1, H:2 * H],
                                r2(msg_b1[l + 1]))
    else:
      sums = _final_call(feat, p, upd_W1[l], r2(upd_b1[l]),
                         upd_W2[l], r2(upd_b2[l]),
                         out1_W, r2(out1_b), out2_W, r2(out2_b))

  return _readout_call(sums, ro1_W, r2(ro1_b), ro2_W, r2(ro2_b))
